# baseline (speedup 1.0000x reference)
import functools

import jax
import jax.numpy as jnp
from jax import lax
from jax.experimental import pallas as pl
from jax.experimental.pallas import tpu as pltpu
from jax.experimental.pallas import tpu_sc as plsc

_GAMMA = 12.0
_EPS = 1e-12

_B = 1024
_NEG = 200
_D = 64
_NC = 2
_NS = 16
_NW = _NC * _NS
_NB = _B // _NW
_NEG_PAD = 208
_NCHUNK = _NEG_PAD // 16


def _sc_body(hidx_hbm, ridx_hbm, neg_hbm, ent_hbm, rel_hbm,
             score_hbm, head_hbm, tail_hbm,
             hidx_v, ridx_v, nidx_v, head_v, relv_v, hr_v, tail_v, score_v,
             sem_h, sem_r, sem_t):
    cid = lax.axis_index("c")
    sid = lax.axis_index("s")
    w = sid * _NC + cid
    b0 = w * _NB

    pltpu.sync_copy(hidx_hbm.at[pl.ds(b0, _NB)], hidx_v)
    pltpu.sync_copy(ridx_hbm.at[pl.ds(b0, _NB)], ridx_v)
    pltpu.sync_copy(neg_hbm.at[pl.ds(b0, _NB)], nidx_v)

    cph = pltpu.async_copy(ent_hbm.at[hidx_v], head_v, sem_h)
    cpr = pltpu.async_copy(rel_hbm.at[ridx_v], relv_v, sem_r)
    cph.wait()
    cpr.wait()

    pltpu.sync_copy(head_v, head_hbm.at[pl.ds(b0, _NB)])

    for i in range(_NB):
        for j4 in range(_D // 16):
            sl = pl.ds(j4 * 16, 16)
            hr_v[i, sl] = head_v[i, sl] + relv_v[i, sl]

    iota = lax.iota(jnp.int32, 16)

    def row_body(i, _):
        cpt = pltpu.async_copy(ent_hbm.at[nidx_v.at[i]],
                               tail_v.at[pl.ds(0, _NEG)], sem_t)
        cpt.wait()

        hr_vecs = [hr_v[i, pl.ds(j4 * 16, 16)] for j4 in range(_D // 16)]

        def chunk_body(ci, _):
            rows = ci * 16 + iota
            acc = jnp.zeros((16,), jnp.float32)
            for j in range(_D):
                tv = plsc.load_gather(
                    tail_v, [rows, jnp.full((16,), j, jnp.int32)])
                d = hr_vecs[j // 16][j % 16] - tv
                acc = acc + d * d
            x = acc + _EPS
            y = plsc.bitcast(
                jnp.int32(0x5F3759DF) - (plsc.bitcast(x, jnp.int32) >> 1),
                jnp.float32)
            for _unused in range(3):
                y = y * (1.5 - 0.5 * x * y * y)
            score_v[pl.ds(ci * 16, 16)] = _GAMMA - x * y
            return 0

        lax.fori_loop(0, _NCHUNK, chunk_body, 0)

        pltpu.sync_copy(score_v.at[pl.ds(0, _NEG)], score_hbm.at[b0 + i])
        pltpu.sync_copy(tail_v.at[pl.ds(0, _NEG)], tail_hbm.at[b0 + i])
        return 0

    lax.fori_loop(0, _NB, row_body, 0)


@jax.jit
def kernel(positive_sample, negative_sample, entity_embedding,
           relation_embedding):
    head_idx = positive_sample[:, 0]
    rel_idx = positive_sample[:, 1]

    mesh = plsc.VectorSubcoreMesh(core_axis_name="c", subcore_axis_name="s")
    score, head, tail = pl.kernel(
        _sc_body,
        out_type=(
            jax.ShapeDtypeStruct((_B, _NEG), jnp.float32),
            jax.ShapeDtypeStruct((_B, _D), jnp.float32),
            jax.ShapeDtypeStruct((_B, _NEG, _D), jnp.float32),
        ),
        mesh=mesh,
        compiler_params=pltpu.CompilerParams(needs_layout_passes=False,
                                             use_tc_tiling_on_sc=False),
        scratch_types=[
            pltpu.VMEM((_NB,), jnp.int32),
            pltpu.VMEM((_NB,), jnp.int32),
            pltpu.VMEM((_NB, _NEG), jnp.int32),
            pltpu.VMEM((_NB, _D), jnp.float32),
            pltpu.VMEM((_NB, _D), jnp.float32),
            pltpu.VMEM((_NB, _D), jnp.float32),
            pltpu.VMEM((_NEG_PAD, _D), jnp.float32),
            pltpu.VMEM((_NEG_PAD,), jnp.float32),
            pltpu.SemaphoreType.DMA,
            pltpu.SemaphoreType.DMA,
            pltpu.SemaphoreType.DMA,
        ],
    )(head_idx, rel_idx, negative_sample, entity_embedding,
      relation_embedding)
    return (score, head[:, None, :], tail)
